# baseline (speedup 1.0000x reference)
import functools

import jax
import jax.numpy as jnp
from jax import lax
from jax.experimental import pallas as pl
from jax.experimental.pallas import tpu as pltpu
from jax.experimental.pallas import tpu_sc as plsc

B, D, L, V, E = 16384, 13, 20, 1000000, 64

NC, NS, LANES = 2, 16, 16
NW = NC * NS
RPT = B // NW
IPT = RPT * L
CB = 16
CROWS = CB * L
NCH = RPT // CB


CIN = 16384
CGRID = (V + CIN - 1) // CIN
CSH = 14


def _conv_body(tt_ref, out_ref):
    eye = jnp.eye(E, dtype=jnp.bfloat16)
    ttb = tt_ref[...].astype(jnp.bfloat16)
    t = lax.dot_general(ttb, eye, (((0,), (0,)), ((), ())),
                        preferred_element_type=jnp.float32)
    u = lax.bitcast_convert_type(t, jnp.uint32)
    p = lax.bitcast_convert_type(
        (u[:, :E // 2] >> 16) | (u[:, E // 2:] & jnp.uint32(0xFFFF0000)),
        jnp.float32)
    q = CIN // 4
    out_ref[...] = jnp.concatenate(
        [p[0:q], p[q:2 * q], p[2 * q:3 * q], p[3 * q:]], axis=1)


def _convert_table(table_t):
    return pl.pallas_call(
        _conv_body,
        grid=(CGRID,),
        in_specs=[pl.BlockSpec((E, CIN), lambda j: (0, j))],
        out_specs=pl.BlockSpec((CIN // 4, 2 * E), lambda j: (j, 0)),
        out_shape=jax.ShapeDtypeStruct((CGRID * CIN // 4, 2 * E), jnp.float32),
    )(table_t)


def _sc_pool_body(cat_hbm, table_hbm, out_hbm, idx_v, m_v, rows_a, rows_b,
                  out_v, sem_a, sem_b):
    wid = lax.axis_index("s") * NC + lax.axis_index("c")
    pltpu.sync_copy(cat_hbm.at[pl.ds(wid * IPT, IPT)], idx_v)

    def prep(s, _):
        ids = idx_v[pl.ds(s * LANES, LANES)]
        m_v[pl.ds(s * LANES, LANES)] = (
            jnp.left_shift(jnp.right_shift(ids, CSH), CSH)
            | jnp.left_shift(jnp.bitwise_and(ids, CIN // 4 - 1), 2)
            | jnp.bitwise_and(jnp.right_shift(ids, CSH - 2), 3)
        )
        return 0

    lax.fori_loop(0, IPT // LANES, prep, 0)

    def start(c, buf, s):
        return pltpu.async_copy(
            table_hbm.at[m_v.at[pl.ds(c * CROWS, CROWS)]], buf, s)

    def accum(c, rows_v):
        def row_body(b, _):
            acc = [[None, None], [None, None]]
            for l in range(L):
                for j in range(2):
                    x = plsc.bitcast(
                        rows_v[b * L + l, pl.ds(j * LANES, LANES)], jnp.bfloat16)
                    a, h = plsc.unpack(x, format=plsc.PackFormat.INTERLEAVED)
                    if l == 0:
                        acc[j][0], acc[j][1] = a, h
                    else:
                        acc[j][0] = acc[j][0] + a
                        acc[j][1] = acc[j][1] + h
            for j in range(2):
                out_v[c * CB + b, pl.ds(j * LANES, LANES)] = \
                    acc[j][0] * (1.0 / L)
                out_v[c * CB + b, pl.ds(2 * LANES + j * LANES, LANES)] = \
                    acc[j][1] * (1.0 / L)
            return 0

        lax.fori_loop(0, CB, row_body, 0)

    bufs, sems = (rows_a, rows_b), (sem_a, sem_b)
    handles = [start(0, bufs[0], sems[0]), None]
    for c in range(NCH):
        if c + 1 < NCH:
            handles[(c + 1) % 2] = start(c + 1, bufs[(c + 1) % 2],
                                         sems[(c + 1) % 2])
        handles[c % 2].wait()
        accum(c, bufs[c % 2])
    pltpu.sync_copy(out_v, out_hbm.at[pl.ds(wid * RPT, RPT)])


def _make_sc_pool():
    mesh = plsc.VectorSubcoreMesh(core_axis_name="c", subcore_axis_name="s")
    return functools.partial(
        pl.kernel,
        mesh=mesh,
        out_type=jax.ShapeDtypeStruct((B, E), jnp.float32),
        scratch_types=[
            pltpu.VMEM((IPT,), jnp.int32),
            pltpu.VMEM((IPT,), jnp.int32),
            pltpu.VMEM((CROWS, E // 2), jnp.float32),
            pltpu.VMEM((CROWS, E // 2), jnp.float32),
            pltpu.VMEM((RPT, E), jnp.float32),
            pltpu.SemaphoreType.DMA,
            pltpu.SemaphoreType.DMA,
        ],
        compiler_params=pltpu.CompilerParams(
            use_tc_tiling_on_sc=False, needs_layout_passes=False),
    )(_sc_pool_body)


_sc_pool = _make_sc_pool()


BLK = 1024
NBLK = B // BLK


def _bottom_body(x_ref, w0_ref, b0_ref, w1_ref, b1_ref, w2_ref, b2_ref, out_ref):
    h = jnp.dot(x_ref[...], w0_ref[...], preferred_element_type=jnp.float32)
    h = jnp.maximum(h + b0_ref[...], 0.0)
    h = jnp.dot(h, w1_ref[...], preferred_element_type=jnp.float32)
    h = jnp.maximum(h + b1_ref[...], 0.0)
    h = jnp.dot(h, w2_ref[...], preferred_element_type=jnp.float32)
    out_ref[...] = jnp.maximum(h + b2_ref[...], 0.0)


def _top_body(de_ref, se_ref, wt0a_ref, wt0b_ref, bt0_ref, wt1_ref, bt1_ref,
              wt2_ref, bt2_ref, out_ref):
    de = de_ref[...]
    se = se_ref[...].astype(jnp.float32)
    inter = jnp.sum(de * se, axis=1, keepdims=True)
    t = jnp.dot(de.astype(jnp.bfloat16), wt0a_ref[...].astype(jnp.bfloat16),
                preferred_element_type=jnp.float32)
    t = jnp.maximum(t + inter * wt0b_ref[...] + bt0_ref[...], 0.0)
    t = jnp.dot(t.astype(jnp.bfloat16), wt1_ref[...].astype(jnp.bfloat16),
                preferred_element_type=jnp.float32)
    t = jnp.maximum(t + bt1_ref[...], 0.0)
    t = jnp.dot(t, wt2_ref[...], preferred_element_type=jnp.float32)
    out_ref[...] = (t + bt2_ref[...])[:, 0]


def _full_spec(shape):
    return pl.BlockSpec(shape, lambda i: (0,) * len(shape))


def _bottom_mlp(x, w0, b0, w1, b1, w2, b2):
    return pl.pallas_call(
        _bottom_body,
        grid=(NBLK,),
        in_specs=[
            pl.BlockSpec((BLK, D), lambda i: (i, 0)),
            _full_spec((D, 512)), _full_spec((1, 512)),
            _full_spec((512, 256)), _full_spec((1, 256)),
            _full_spec((256, E)), _full_spec((1, E)),
        ],
        out_specs=pl.BlockSpec((BLK, E), lambda i: (i, 0)),
        out_shape=jax.ShapeDtypeStruct((B, E), jnp.float32),
    )(x, w0, b0, w1, b1, w2, b2)


def _top_mlp(dense_emb, sparse_emb, wt0a, wt0b, bt0, wt1, bt1, wt2, bt2):
    return pl.pallas_call(
        _top_body,
        grid=(NBLK,),
        in_specs=[
            pl.BlockSpec((BLK, E), lambda i: (i, 0)),
            pl.BlockSpec((BLK, E), lambda i: (i, 0)),
            _full_spec((E, 512)), _full_spec((1, 512)), _full_spec((1, 512)),
            _full_spec((512, 256)), _full_spec((1, 256)),
            _full_spec((256, 1)), _full_spec((1, 1)),
        ],
        out_specs=pl.BlockSpec((BLK,), lambda i: (i,)),
        out_shape=jax.ShapeDtypeStruct((B,), jnp.float32),
    )(dense_emb, sparse_emb, wt0a, wt0b, bt0, wt1, bt1, wt2, bt2)


def kernel(dense_features, category_ids, W0, b0, W1, b1, W2, b2, emb_table,
           Wt0, bt0, Wt1, bt1, Wt2, bt2):
    cat_flat = category_ids.reshape(-1)
    table_t, cat_flat = lax.optimization_barrier(
        (jnp.transpose(emb_table), cat_flat))
    t128 = _convert_table(table_t)
    t_lin = t128.reshape(CGRID * CIN, E // 2)
    sparse_emb = _sc_pool(cat_flat, t_lin)
    dense_emb = _bottom_mlp(
        dense_features, W0, b0.reshape(1, -1), W1, b1.reshape(1, -1),
        W2, b2.reshape(1, -1))
    return _top_mlp(
        dense_emb, sparse_emb, Wt0[:E], Wt0[E:E + 1], bt0.reshape(1, -1),
        Wt1, bt1.reshape(1, -1), Wt2, bt2.reshape(1, -1))


# speedup vs baseline: 1.0214x; 1.0214x over previous
import functools

import jax
import jax.numpy as jnp
from jax import lax
from jax.experimental import pallas as pl
from jax.experimental.pallas import tpu as pltpu
from jax.experimental.pallas import tpu_sc as plsc

B, D, L, V, E = 16384, 13, 20, 1000000, 64

NC, NS, LANES = 2, 16, 16
NW = NC * NS
RPT = B // NW
IPT = RPT * L
CB = 16
CROWS = CB * L
NCH = RPT // CB


CIN = 16384
CGRID = (V + CIN - 1) // CIN
CSH = 14


def _conv_body(tt_ref, out_ref):
    eye = jnp.eye(E, dtype=jnp.bfloat16)
    ttb = tt_ref[...].astype(jnp.bfloat16)
    t = lax.dot_general(ttb, eye, (((0,), (0,)), ((), ())),
                        preferred_element_type=jnp.float32)
    u = lax.bitcast_convert_type(t, jnp.uint32)
    p = lax.bitcast_convert_type(
        (u[:, :E // 2] >> 16) | (u[:, E // 2:] & jnp.uint32(0xFFFF0000)),
        jnp.float32)
    q = CIN // 4
    out_ref[...] = jnp.concatenate(
        [p[0:q], p[q:2 * q], p[2 * q:3 * q], p[3 * q:]], axis=1)


def _convert_table(table_t):
    return pl.pallas_call(
        _conv_body,
        grid=(CGRID,),
        in_specs=[pl.BlockSpec((E, CIN), lambda j: (0, j))],
        out_specs=pl.BlockSpec((CIN // 4, 2 * E), lambda j: (j, 0)),
        out_shape=jax.ShapeDtypeStruct((CGRID * CIN // 4, 2 * E), jnp.float32),
    )(table_t)


NBUF = 4


def _sc_pool_body(cat_hbm, table_hbm, out_hbm, idx_v, m_v, rows_a, rows_b,
                  rows_c, rows_d, out_v, sem_a, sem_b, sem_c, sem_d):
    wid = lax.axis_index("s") * NC + lax.axis_index("c")
    pltpu.sync_copy(cat_hbm.at[pl.ds(wid * IPT, IPT)], idx_v)

    def prep(s, _):
        ids = idx_v[pl.ds(s * LANES, LANES)]
        m_v[pl.ds(s * LANES, LANES)] = (
            jnp.left_shift(jnp.right_shift(ids, CSH), CSH)
            | jnp.left_shift(jnp.bitwise_and(ids, CIN // 4 - 1), 2)
            | jnp.bitwise_and(jnp.right_shift(ids, CSH - 2), 3)
        )
        return 0

    lax.fori_loop(0, IPT // LANES, prep, 0)

    def start(c, buf, s):
        return pltpu.async_copy(
            table_hbm.at[m_v.at[pl.ds(c * CROWS, CROWS)]], buf, s)

    def accum(c, rows_v):
        def row_body(b, _):
            acc = [[None, None], [None, None]]
            for l in range(L):
                for j in range(2):
                    x = plsc.bitcast(
                        rows_v[b * L + l, pl.ds(j * LANES, LANES)], jnp.bfloat16)
                    a, h = plsc.unpack(x, format=plsc.PackFormat.INTERLEAVED)
                    if l == 0:
                        acc[j][0], acc[j][1] = a, h
                    else:
                        acc[j][0] = acc[j][0] + a
                        acc[j][1] = acc[j][1] + h
            for j in range(2):
                out_v[c * CB + b, pl.ds(j * LANES, LANES)] = \
                    acc[j][0] * (1.0 / L)
                out_v[c * CB + b, pl.ds(2 * LANES + j * LANES, LANES)] = \
                    acc[j][1] * (1.0 / L)
            return 0

        lax.fori_loop(0, CB, row_body, 0)

    bufs = (rows_a, rows_b, rows_c, rows_d)
    sems = (sem_a, sem_b, sem_c, sem_d)
    handles = [None] * NBUF
    for c in range(NBUF - 1):
        handles[c] = start(c, bufs[c], sems[c])
    for c in range(NCH):
        nxt = c + NBUF - 1
        if nxt < NCH:
            handles[nxt % NBUF] = start(nxt, bufs[nxt % NBUF], sems[nxt % NBUF])
        handles[c % NBUF].wait()
        accum(c, bufs[c % NBUF])
    pltpu.sync_copy(out_v, out_hbm.at[pl.ds(wid * RPT, RPT)])


def _make_sc_pool():
    mesh = plsc.VectorSubcoreMesh(core_axis_name="c", subcore_axis_name="s")
    return functools.partial(
        pl.kernel,
        mesh=mesh,
        out_type=jax.ShapeDtypeStruct((B, E), jnp.float32),
        scratch_types=[
            pltpu.VMEM((IPT,), jnp.int32),
            pltpu.VMEM((IPT,), jnp.int32),
            pltpu.VMEM((CROWS, E // 2), jnp.float32),
            pltpu.VMEM((CROWS, E // 2), jnp.float32),
            pltpu.VMEM((CROWS, E // 2), jnp.float32),
            pltpu.VMEM((CROWS, E // 2), jnp.float32),
            pltpu.VMEM((RPT, E), jnp.float32),
            pltpu.SemaphoreType.DMA,
            pltpu.SemaphoreType.DMA,
            pltpu.SemaphoreType.DMA,
            pltpu.SemaphoreType.DMA,
        ],
        compiler_params=pltpu.CompilerParams(
            use_tc_tiling_on_sc=False, needs_layout_passes=False),
    )(_sc_pool_body)


_sc_pool = _make_sc_pool()


BLK = 1024
NBLK = B // BLK


def _bottom_body(x_ref, w0_ref, b0_ref, w1_ref, b1_ref, w2_ref, b2_ref, out_ref):
    h = jnp.dot(x_ref[...], w0_ref[...], preferred_element_type=jnp.float32)
    h = jnp.maximum(h + b0_ref[...], 0.0)
    h = jnp.dot(h, w1_ref[...], preferred_element_type=jnp.float32)
    h = jnp.maximum(h + b1_ref[...], 0.0)
    h = jnp.dot(h, w2_ref[...], preferred_element_type=jnp.float32)
    out_ref[...] = jnp.maximum(h + b2_ref[...], 0.0)


def _top_body(de_ref, se_ref, wt0a_ref, wt0b_ref, bt0_ref, wt1_ref, bt1_ref,
              wt2_ref, bt2_ref, out_ref):
    de = de_ref[...]
    se = se_ref[...].astype(jnp.float32)
    inter = jnp.sum(de * se, axis=1, keepdims=True)
    t = jnp.dot(de, wt0a_ref[...], preferred_element_type=jnp.float32)
    t = jnp.maximum(t + inter * wt0b_ref[...] + bt0_ref[...], 0.0)
    t = jnp.dot(t, wt1_ref[...], preferred_element_type=jnp.float32)
    t = jnp.maximum(t + bt1_ref[...], 0.0)
    t = jnp.dot(t, wt2_ref[...], preferred_element_type=jnp.float32)
    out_ref[...] = (t + bt2_ref[...])[:, 0]


def _full_spec(shape):
    return pl.BlockSpec(shape, lambda i: (0,) * len(shape))


def _bottom_mlp(x, w0, b0, w1, b1, w2, b2):
    return pl.pallas_call(
        _bottom_body,
        grid=(NBLK,),
        in_specs=[
            pl.BlockSpec((BLK, D), lambda i: (i, 0)),
            _full_spec((D, 512)), _full_spec((1, 512)),
            _full_spec((512, 256)), _full_spec((1, 256)),
            _full_spec((256, E)), _full_spec((1, E)),
        ],
        out_specs=pl.BlockSpec((BLK, E), lambda i: (i, 0)),
        out_shape=jax.ShapeDtypeStruct((B, E), jnp.float32),
    )(x, w0, b0, w1, b1, w2, b2)


def _top_mlp(dense_emb, sparse_emb, wt0a, wt0b, bt0, wt1, bt1, wt2, bt2):
    return pl.pallas_call(
        _top_body,
        grid=(NBLK,),
        in_specs=[
            pl.BlockSpec((BLK, E), lambda i: (i, 0)),
            pl.BlockSpec((BLK, E), lambda i: (i, 0)),
            _full_spec((E, 512)), _full_spec((1, 512)), _full_spec((1, 512)),
            _full_spec((512, 256)), _full_spec((1, 256)),
            _full_spec((256, 1)), _full_spec((1, 1)),
        ],
        out_specs=pl.BlockSpec((BLK,), lambda i: (i,)),
        out_shape=jax.ShapeDtypeStruct((B,), jnp.float32),
    )(dense_emb, sparse_emb, wt0a, wt0b, bt0, wt1, bt1, wt2, bt2)


def kernel(dense_features, category_ids, W0, b0, W1, b1, W2, b2, emb_table,
           Wt0, bt0, Wt1, bt1, Wt2, bt2):
    cat_flat = category_ids.reshape(-1)
    table_t, cat_flat = lax.optimization_barrier(
        (jnp.transpose(emb_table), cat_flat))
    t128 = _convert_table(table_t)
    t_lin = t128.reshape(CGRID * CIN, E // 2)
    sparse_emb = _sc_pool(cat_flat, t_lin)
    dense_emb = _bottom_mlp(
        dense_features, W0, b0.reshape(1, -1), W1, b1.reshape(1, -1),
        W2, b2.reshape(1, -1))
    return _top_mlp(
        dense_emb, sparse_emb, Wt0[:E], Wt0[E:E + 1], bt0.reshape(1, -1),
        Wt1, bt1.reshape(1, -1), Wt2, bt2.reshape(1, -1))


# speedup vs baseline: 1.0277x; 1.0061x over previous
import functools

import jax
import jax.numpy as jnp
from jax import lax
from jax.experimental import pallas as pl
from jax.experimental.pallas import tpu as pltpu
from jax.experimental.pallas import tpu_sc as plsc

B, D, L, V, E = 16384, 13, 20, 1000000, 64

NC, NS, LANES = 2, 16, 16
NW = NC * NS
RPT = B // NW
IPT = RPT * L
CB = 16
CROWS = CB * L
NCH = RPT // CB


CIN = 32768
CGRID = (V + CIN - 1) // CIN
CSH = 15


def _conv_body(tt_ref, out_ref):
    eye = jnp.eye(E, dtype=jnp.bfloat16)
    ttb = tt_ref[...].astype(jnp.bfloat16)
    t = lax.dot_general(ttb, eye, (((0,), (0,)), ((), ())),
                        preferred_element_type=jnp.float32)
    u = lax.bitcast_convert_type(t, jnp.uint32)
    p = lax.bitcast_convert_type(
        (u[:, :E // 2] >> 16) | (u[:, E // 2:] & jnp.uint32(0xFFFF0000)),
        jnp.float32)
    q = CIN // 4
    out_ref[...] = jnp.concatenate(
        [p[0:q], p[q:2 * q], p[2 * q:3 * q], p[3 * q:]], axis=1)


def _convert_table(table_t):
    return pl.pallas_call(
        _conv_body,
        grid=(CGRID,),
        in_specs=[pl.BlockSpec((E, CIN), lambda j: (0, j))],
        out_specs=pl.BlockSpec((CIN // 4, 2 * E), lambda j: (j, 0)),
        out_shape=jax.ShapeDtypeStruct((CGRID * CIN // 4, 2 * E), jnp.float32),
    )(table_t)


NBUF = 6


def _sc_pool_body(cat_hbm, table_hbm, out_hbm, idx_v, m_v, rows_a, rows_b,
                  rows_c, rows_d, rows_e, rows_f, out_v,
                  sem_a, sem_b, sem_c, sem_d, sem_e, sem_f):
    wid = lax.axis_index("s") * NC + lax.axis_index("c")
    pltpu.sync_copy(cat_hbm.at[pl.ds(wid * IPT, IPT)], idx_v)

    def prep(s, _):
        ids = idx_v[pl.ds(s * LANES, LANES)]
        m_v[pl.ds(s * LANES, LANES)] = (
            jnp.left_shift(jnp.right_shift(ids, CSH), CSH)
            | jnp.left_shift(jnp.bitwise_and(ids, CIN // 4 - 1), 2)
            | jnp.bitwise_and(jnp.right_shift(ids, CSH - 2), 3)
        )
        return 0

    lax.fori_loop(0, IPT // LANES, prep, 0)

    def start(c, buf, s):
        return pltpu.async_copy(
            table_hbm.at[m_v.at[pl.ds(c * CROWS, CROWS)]], buf, s)

    def accum(c, rows_v):
        def row_body(b, _):
            acc = [[None, None], [None, None]]
            for l in range(L):
                for j in range(2):
                    x = plsc.bitcast(
                        rows_v[b * L + l, pl.ds(j * LANES, LANES)], jnp.bfloat16)
                    a, h = plsc.unpack(x, format=plsc.PackFormat.INTERLEAVED)
                    if l == 0:
                        acc[j][0], acc[j][1] = a, h
                    else:
                        acc[j][0] = acc[j][0] + a
                        acc[j][1] = acc[j][1] + h
            for j in range(2):
                out_v[c * CB + b, pl.ds(j * LANES, LANES)] = \
                    acc[j][0] * (1.0 / L)
                out_v[c * CB + b, pl.ds(2 * LANES + j * LANES, LANES)] = \
                    acc[j][1] * (1.0 / L)
            return 0

        lax.fori_loop(0, CB, row_body, 0)

    bufs = (rows_a, rows_b, rows_c, rows_d, rows_e, rows_f)
    sems = (sem_a, sem_b, sem_c, sem_d, sem_e, sem_f)
    handles = [None] * NBUF
    for c in range(NBUF - 1):
        handles[c] = start(c, bufs[c], sems[c])
    for c in range(NCH):
        nxt = c + NBUF - 1
        if nxt < NCH:
            handles[nxt % NBUF] = start(nxt, bufs[nxt % NBUF], sems[nxt % NBUF])
        handles[c % NBUF].wait()
        accum(c, bufs[c % NBUF])
    pltpu.sync_copy(out_v, out_hbm.at[pl.ds(wid * RPT, RPT)])


def _make_sc_pool():
    mesh = plsc.VectorSubcoreMesh(core_axis_name="c", subcore_axis_name="s")
    return functools.partial(
        pl.kernel,
        mesh=mesh,
        out_type=jax.ShapeDtypeStruct((B, E), jnp.float32),
        scratch_types=[
            pltpu.VMEM((IPT,), jnp.int32),
            pltpu.VMEM((IPT,), jnp.int32),
            pltpu.VMEM((CROWS, E // 2), jnp.float32),
            pltpu.VMEM((CROWS, E // 2), jnp.float32),
            pltpu.VMEM((CROWS, E // 2), jnp.float32),
            pltpu.VMEM((CROWS, E // 2), jnp.float32),
            pltpu.VMEM((CROWS, E // 2), jnp.float32),
            pltpu.VMEM((CROWS, E // 2), jnp.float32),
            pltpu.VMEM((RPT, E), jnp.float32),
            pltpu.SemaphoreType.DMA,
            pltpu.SemaphoreType.DMA,
            pltpu.SemaphoreType.DMA,
            pltpu.SemaphoreType.DMA,
            pltpu.SemaphoreType.DMA,
            pltpu.SemaphoreType.DMA,
        ],
        compiler_params=pltpu.CompilerParams(
            use_tc_tiling_on_sc=False, needs_layout_passes=False),
    )(_sc_pool_body)


_sc_pool = _make_sc_pool()


BLK = 1024
NBLK = B // BLK


def _bottom_body(x_ref, w0_ref, b0_ref, w1_ref, b1_ref, w2_ref, b2_ref, out_ref):
    h = jnp.dot(x_ref[...], w0_ref[...], preferred_element_type=jnp.float32)
    h = jnp.maximum(h + b0_ref[...], 0.0)
    h = jnp.dot(h, w1_ref[...], preferred_element_type=jnp.float32)
    h = jnp.maximum(h + b1_ref[...], 0.0)
    h = jnp.dot(h, w2_ref[...], preferred_element_type=jnp.float32)
    out_ref[...] = jnp.maximum(h + b2_ref[...], 0.0)


def _top_body(de_ref, se_ref, wt0a_ref, wt0b_ref, bt0_ref, wt1_ref, bt1_ref,
              wt2_ref, bt2_ref, out_ref):
    de = de_ref[...]
    se = se_ref[...].astype(jnp.float32)
    inter = jnp.sum(de * se, axis=1, keepdims=True)
    t = jnp.dot(de, wt0a_ref[...], preferred_element_type=jnp.float32)
    t = jnp.maximum(t + inter * wt0b_ref[...] + bt0_ref[...], 0.0)
    t = jnp.dot(t, wt1_ref[...], preferred_element_type=jnp.float32)
    t = jnp.maximum(t + bt1_ref[...], 0.0)
    t = jnp.dot(t, wt2_ref[...], preferred_element_type=jnp.float32)
    out_ref[...] = (t + bt2_ref[...])[:, 0]


def _full_spec(shape):
    return pl.BlockSpec(shape, lambda i: (0,) * len(shape))


def _bottom_mlp(x, w0, b0, w1, b1, w2, b2):
    return pl.pallas_call(
        _bottom_body,
        grid=(NBLK,),
        in_specs=[
            pl.BlockSpec((BLK, D), lambda i: (i, 0)),
            _full_spec((D, 512)), _full_spec((1, 512)),
            _full_spec((512, 256)), _full_spec((1, 256)),
            _full_spec((256, E)), _full_spec((1, E)),
        ],
        out_specs=pl.BlockSpec((BLK, E), lambda i: (i, 0)),
        out_shape=jax.ShapeDtypeStruct((B, E), jnp.float32),
    )(x, w0, b0, w1, b1, w2, b2)


def _top_mlp(dense_emb, sparse_emb, wt0a, wt0b, bt0, wt1, bt1, wt2, bt2):
    return pl.pallas_call(
        _top_body,
        grid=(NBLK,),
        in_specs=[
            pl.BlockSpec((BLK, E), lambda i: (i, 0)),
            pl.BlockSpec((BLK, E), lambda i: (i, 0)),
            _full_spec((E, 512)), _full_spec((1, 512)), _full_spec((1, 512)),
            _full_spec((512, 256)), _full_spec((1, 256)),
            _full_spec((256, 1)), _full_spec((1, 1)),
        ],
        out_specs=pl.BlockSpec((BLK,), lambda i: (i,)),
        out_shape=jax.ShapeDtypeStruct((B,), jnp.float32),
    )(dense_emb, sparse_emb, wt0a, wt0b, bt0, wt1, bt1, wt2, bt2)


def kernel(dense_features, category_ids, W0, b0, W1, b1, W2, b2, emb_table,
           Wt0, bt0, Wt1, bt1, Wt2, bt2):
    cat_flat = category_ids.reshape(-1)
    table_t, cat_flat = lax.optimization_barrier(
        (jnp.transpose(emb_table), cat_flat))
    t128 = _convert_table(table_t)
    t_lin = t128.reshape(CGRID * CIN, E // 2)
    sparse_emb = _sc_pool(cat_flat, t_lin)
    dense_emb = _bottom_mlp(
        dense_features, W0, b0.reshape(1, -1), W1, b1.reshape(1, -1),
        W2, b2.reshape(1, -1))
    return _top_mlp(
        dense_emb, sparse_emb, Wt0[:E], Wt0[E:E + 1], bt0.reshape(1, -1),
        Wt1, bt1.reshape(1, -1), Wt2, bt2.reshape(1, -1))
